# baseline (speedup 1.0000x reference)
import functools

import jax
import jax.numpy as jnp
from jax import lax
from jax.experimental import pallas as pl
from jax.experimental.pallas import tpu as pltpu
from jax.experimental.pallas import tpu_sc as plsc

NC = 2
NS = 16
LANES = 16
NW = NC * NS


def _tc_pack_tables(features, w1lo, w1hi, w2lo, w2hi):
    N, D = features.shape
    H = w1lo.shape[1]
    BN = 1000

    def pack(ylo, yhi):
        lo16 = jax.lax.bitcast_convert_type(
            ylo.astype(jnp.bfloat16), jnp.uint16).astype(jnp.uint32)
        hi16 = jax.lax.bitcast_convert_type(
            yhi.astype(jnp.bfloat16), jnp.uint16).astype(jnp.uint32)
        return jax.lax.bitcast_convert_type(lo16 | (hi16 << 16), jnp.float32)

    def body(f_ref, w1l_ref, w1h_ref, w2l_ref, w2h_ref, q_ref, p_ref):
        x = f_ref[...].astype(jnp.bfloat16)
        q_ref[...] = pack(
            jnp.dot(x, w1l_ref[...], preferred_element_type=jnp.float32),
            jnp.dot(x, w1h_ref[...], preferred_element_type=jnp.float32))
        p_ref[...] = pack(
            jnp.dot(x, w2l_ref[...], preferred_element_type=jnp.float32),
            jnp.dot(x, w2h_ref[...], preferred_element_type=jnp.float32))

    wspec = pl.BlockSpec((D, H), lambda i: (0, 0))
    return pl.pallas_call(
        body,
        grid=(N // BN,),
        in_specs=[pl.BlockSpec((BN, D), lambda i: (i, 0)),
                  wspec, wspec, wspec, wspec],
        out_specs=(pl.BlockSpec((BN, H), lambda i: (i, 0)),
                   pl.BlockSpec((BN, H), lambda i: (i, 0))),
        out_shape=(jax.ShapeDtypeStruct((N, H), jnp.float32),
                   jax.ShapeDtypeStruct((N, H), jnp.float32)),
    )(features, w1lo, w1hi, w2lo, w2hi)


def _sc_gather_mean(nodes, neigh2d, q_packed, p_packed, B, DEG, D_OUT):
    b_per_w = B // NW
    CH = 128 // DEG
    n_chunks = b_per_w // CH
    n_self = b_per_w // 128
    inv_deg = 1.0 / DEG
    H = D_OUT // 2
    nk = H // LANES

    mesh = plsc.VectorSubcoreMesh(
        core_axis_name="c", subcore_axis_name="s",
        num_cores=NC, num_subcores=NS)

    @functools.partial(
        pl.kernel,
        out_type=(
            jax.ShapeDtypeStruct((B, H), jnp.float32),
            jax.ShapeDtypeStruct((B, H), jnp.float32),
        ),
        mesh=mesh,
        scratch_types=[
            pltpu.VMEM((n_chunks, CH * DEG), jnp.int32),
            pltpu.VMEM((b_per_w,), jnp.int32),
            pltpu.VMEM((2, CH * DEG, H), jnp.float32),
            pltpu.VMEM((128, H), jnp.float32),
            pltpu.VMEM((CH, H), jnp.float32),
            pltpu.SemaphoreType.DMA,
            pltpu.SemaphoreType.DMA,
            pltpu.SemaphoreType.DMA,
        ],
    )
    def sc_kernel(nodes_hbm, nidx_hbm, qk_hbm, pk_hbm,
                  selfout_hbm, meanout_hbm,
                  idx_all, sidx, rows2, srows, acc_v, sem0, sem1, ssem):
        wid = lax.axis_index("s") * NC + lax.axis_index("c")
        base = wid * b_per_w
        cbase = wid * n_chunks

        pltpu.sync_copy(nidx_hbm.at[pl.ds(cbase, n_chunks)], idx_all)
        pltpu.sync_copy(nodes_hbm.at[pl.ds(base, b_per_w)], sidx)

        sems = (sem0, sem1)
        pltpu.async_copy(pk_hbm.at[idx_all.at[0]], rows2.at[0], sem0)
        pltpu.async_copy(pk_hbm.at[idx_all.at[1]], rows2.at[1], sem1)

        @pl.loop(0, n_self)
        def _self_loop(sc):
            pltpu.async_copy(
                qk_hbm.at[sidx.at[pl.ds(sc * 128, 128)]], srows, ssem
            ).wait()
            pltpu.sync_copy(
                srows, selfout_hbm.at[pl.ds(base + sc * 128, 128)])

        mask_hi = jnp.int32(-65536)
        half = jnp.int32(0x8000)

        @pl.loop(0, n_chunks, step=2)
        def _chunk_loop(c):
            for bsel in range(2):
                cc = c + bsel
                pltpu.make_async_copy(
                    pk_hbm.at[idx_all.at[0]], rows2.at[bsel], sems[bsel]
                ).wait()
                for r in range(CH):

                    def unpack_row(j):
                        out = []
                        for k in range(nk):
                            v = jax.lax.bitcast_convert_type(
                                rows2[bsel, j, pl.ds(k * LANES, LANES)],
                                jnp.int32)
                            e = jax.lax.bitcast_convert_type(
                                v << 16, jnp.float32)
                            o = jax.lax.bitcast_convert_type(
                                v & mask_hi, jnp.float32)
                            out.append((e, o))
                        return out

                    init = tuple(unpack_row(r * DEG))

                    @pl.loop(1, DEG, init_carry=init, unroll=4)
                    def _row_loop(j, carry):
                        row = unpack_row(r * DEG + j)
                        return tuple(
                            (ce + e, co + o)
                            for (ce, co), (e, o) in zip(carry, row))

                    for k in range(nk):
                        ce, co = _row_loop[k]
                        ei = jax.lax.bitcast_convert_type(
                            ce * inv_deg, jnp.int32)
                        oi = jax.lax.bitcast_convert_type(
                            co * inv_deg, jnp.int32)
                        lo = jax.lax.shift_right_logical(ei + half, 16)
                        hi = (oi + half) & mask_hi
                        acc_v[r, pl.ds(k * LANES, LANES)] = (
                            jax.lax.bitcast_convert_type(
                                lo | hi, jnp.float32))
                pltpu.sync_copy(
                    acc_v, meanout_hbm.at[pl.ds(base + cc * CH, CH)])

                @pl.when(cc + 2 < n_chunks)
                def _refill():
                    pltpu.async_copy(
                        pk_hbm.at[idx_all.at[cc + 2]], rows2.at[bsel],
                        sems[bsel])

    return sc_kernel(nodes, neigh2d, q_packed, p_packed)


def _tc_combine(self_packed, mean_packed, b2d, D_OUT):
    B, H = self_packed.shape
    BM = 1024

    def unpack_natural(v_f32):
        v = jax.lax.bitcast_convert_type(v_f32, jnp.int32)
        e = jax.lax.bitcast_convert_type(v << 16, jnp.float32)
        o = jax.lax.bitcast_convert_type(v & jnp.int32(-65536), jnp.float32)
        em = e.reshape(BM, H // 16, 16)
        om = o.reshape(BM, H // 16, 16)
        return jnp.concatenate([em, om], axis=2).reshape(BM, 2 * H)

    def body(s_ref, n_ref, b_ref, o_ref):
        x = unpack_natural(s_ref[...]) + unpack_natural(n_ref[...])
        x = x + b_ref[...]
        x = jnp.maximum(x, 0.0)
        nrm = jnp.sqrt(jnp.sum(x * x, axis=1, keepdims=True))
        o_ref[...] = x / jnp.maximum(nrm, 1e-12)

    return pl.pallas_call(
        body,
        grid=(B // BM,),
        in_specs=[
            pl.BlockSpec((BM, H), lambda i: (i, 0)),
            pl.BlockSpec((BM, H), lambda i: (i, 0)),
            pl.BlockSpec((1, D_OUT), lambda i: (0, 0)),
        ],
        out_specs=pl.BlockSpec((BM, D_OUT), lambda i: (i, 0)),
        out_shape=jax.ShapeDtypeStruct((B, D_OUT), jnp.float32),
    )(self_packed, mean_packed, b2d)


def _split_cols(wmat):
    D, D_OUT = wmat.shape
    g = wmat.reshape(D, D_OUT // 32, 2, 16)
    lo = g[:, :, 0, :].reshape(D, D_OUT // 2).astype(jnp.bfloat16)
    hi = g[:, :, 1, :].reshape(D, D_OUT // 2).astype(jnp.bfloat16)
    return lo, hi


def kernel(nodes, neigh_index, features, w, b):
    B, DEG = neigh_index.shape
    N, D = features.shape
    CH = 128 // DEG
    neigh2d = neigh_index.reshape(B // CH, CH * DEG)
    w1lo, w1hi = _split_cols(w[:D])
    w2lo, w2hi = _split_cols(w[D:])
    D_OUT = w.shape[1]
    q_packed, p_packed = _tc_pack_tables(features, w1lo, w1hi, w2lo, w2hi)
    self_packed, mean_packed = _sc_gather_mean(
        nodes, neigh2d, q_packed, p_packed, B, DEG, D_OUT)
    return _tc_combine(self_packed, mean_packed, b.reshape(1, -1), D_OUT)


# speedup vs baseline: 1.3446x; 1.3446x over previous
import functools

import jax
import jax.numpy as jnp
from jax import lax
from jax.experimental import pallas as pl
from jax.experimental.pallas import tpu as pltpu
from jax.experimental.pallas import tpu_sc as plsc

NC = 2
NS = 16
LANES = 16
NW = NC * NS


def _tc_pack_table(features, w2lo_bf, w2hi_bf):
    N, D = features.shape
    H = w2lo_bf.shape[1]
    BN = 1000

    def body(f_ref, wl_ref, wh_ref, o_ref):
        x = f_ref[...].astype(jnp.bfloat16)
        ylo = jnp.dot(x, wl_ref[...], preferred_element_type=jnp.float32)
        yhi = jnp.dot(x, wh_ref[...], preferred_element_type=jnp.float32)
        lo16 = jax.lax.bitcast_convert_type(
            ylo.astype(jnp.bfloat16), jnp.uint16).astype(jnp.uint32)
        hi16 = jax.lax.bitcast_convert_type(
            yhi.astype(jnp.bfloat16), jnp.uint16).astype(jnp.uint32)
        packed = lo16 | (hi16 << 16)
        o_ref[...] = jax.lax.bitcast_convert_type(packed, jnp.float32)

    return pl.pallas_call(
        body,
        grid=(N // BN,),
        in_specs=[
            pl.BlockSpec((BN, D), lambda i: (i, 0)),
            pl.BlockSpec((D, H), lambda i: (0, 0)),
            pl.BlockSpec((D, H), lambda i: (0, 0)),
        ],
        out_specs=pl.BlockSpec((BN, H), lambda i: (i, 0)),
        out_shape=jax.ShapeDtypeStruct((N, H), jnp.float32),
    )(features, w2lo_bf, w2hi_bf)


def _sc_gather_mean(nodes, neigh2d, features, p_packed, B, DEG, D):
    b_per_w = B // NW
    CH = 128 // DEG
    n_chunks = b_per_w // CH
    n_self = b_per_w // 128
    inv_deg = 1.0 / DEG
    H = D // 2
    nk = H // LANES

    mesh = plsc.VectorSubcoreMesh(
        core_axis_name="c", subcore_axis_name="s",
        num_cores=NC, num_subcores=NS)

    @functools.partial(
        pl.kernel,
        out_type=jax.ShapeDtypeStruct((B, D), jnp.float32),
        mesh=mesh,
        scratch_types=[
            pltpu.VMEM((b_per_w,), jnp.int32),
            pltpu.VMEM((128, D), jnp.float32),
            pltpu.SemaphoreType.DMA,
        ],
    )
    def sc_self_kernel(nodes_hbm, feat_hbm, selfout_hbm, sidx, srows, ssem):
        wid = lax.axis_index("s") * NC + lax.axis_index("c")
        base = wid * b_per_w
        pltpu.sync_copy(nodes_hbm.at[pl.ds(base, b_per_w)], sidx)

        @pl.loop(0, n_self)
        def _self_loop(sc):
            pltpu.async_copy(
                feat_hbm.at[sidx.at[pl.ds(sc * 128, 128)]], srows, ssem
            ).wait()
            pltpu.sync_copy(
                srows, selfout_hbm.at[pl.ds(base + sc * 128, 128)])

    @functools.partial(
        pl.kernel,
        out_type=jax.ShapeDtypeStruct((B, D), jnp.float32),
        mesh=mesh,
        scratch_types=[
            pltpu.VMEM((n_chunks, CH * DEG), jnp.int32),
            pltpu.VMEM((2, CH * DEG, H), jnp.float32),
            pltpu.VMEM((CH, D), jnp.float32),
            pltpu.SemaphoreType.DMA,
            pltpu.SemaphoreType.DMA,
        ],
    )
    def sc_kernel(nidx_hbm, pk_hbm, neighout_hbm,
                  idx_all, rows2, acc_v, sem0, sem1):
        wid = lax.axis_index("s") * NC + lax.axis_index("c")
        base = wid * b_per_w
        cbase = wid * n_chunks

        pltpu.sync_copy(nidx_hbm.at[pl.ds(cbase, n_chunks)], idx_all)

        sems = (sem0, sem1)
        pltpu.async_copy(pk_hbm.at[idx_all.at[0]], rows2.at[0], sem0)
        pltpu.async_copy(pk_hbm.at[idx_all.at[1]], rows2.at[1], sem1)

        mask_hi = jnp.int32(-65536)

        @pl.loop(0, n_chunks, step=2)
        def _chunk_loop(c):
            for bsel in range(2):
                cc = c + bsel
                pltpu.make_async_copy(
                    pk_hbm.at[idx_all.at[0]], rows2.at[bsel], sems[bsel]
                ).wait()
                for r in range(CH):

                    def unpack_row(j):
                        out = []
                        for k in range(nk):
                            v = jax.lax.bitcast_convert_type(
                                rows2[bsel, j, pl.ds(k * LANES, LANES)],
                                jnp.int32)
                            e = jax.lax.bitcast_convert_type(
                                v << 16, jnp.float32)
                            o = jax.lax.bitcast_convert_type(
                                v & mask_hi, jnp.float32)
                            out.append((e, o))
                        return out

                    init = tuple(unpack_row(r * DEG))

                    @pl.loop(1, DEG, init_carry=init, unroll=4)
                    def _row_loop(j, carry):
                        row = unpack_row(r * DEG + j)
                        return tuple(
                            (ce + e, co + o)
                            for (ce, co), (e, o) in zip(carry, row))

                    for k in range(nk):
                        ce, co = _row_loop[k]
                        acc_v[r, pl.ds(k * 2 * LANES, LANES)] = ce * inv_deg
                        acc_v[r, pl.ds(k * 2 * LANES + LANES, LANES)] = (
                            co * inv_deg)
                pltpu.sync_copy(
                    acc_v, neighout_hbm.at[pl.ds(base + cc * CH, CH)])

                @pl.when(cc + 2 < n_chunks)
                def _refill():
                    pltpu.async_copy(
                        pk_hbm.at[idx_all.at[cc + 2]], rows2.at[bsel],
                        sems[bsel])

    self_feats = sc_self_kernel(nodes, features)
    neigh_contrib = sc_kernel(neigh2d, p_packed)
    return self_feats, neigh_contrib


def _tc_combine(self_feats, neigh_contrib, w1, b2d):
    B, D = self_feats.shape
    D_OUT = w1.shape[1]
    BM = 1024

    def body(s_ref, n_ref, w1_ref, b_ref, o_ref):
        x = jnp.dot(s_ref[...], w1_ref[...], preferred_element_type=jnp.float32)
        x = x + n_ref[...]
        x = x + b_ref[...]
        x = jnp.maximum(x, 0.0)
        nrm = jnp.sqrt(jnp.sum(x * x, axis=1, keepdims=True))
        o_ref[...] = x / jnp.maximum(nrm, 1e-12)

    return pl.pallas_call(
        body,
        grid=(B // BM,),
        in_specs=[
            pl.BlockSpec((BM, D), lambda i: (i, 0)),
            pl.BlockSpec((BM, D), lambda i: (i, 0)),
            pl.BlockSpec((D, D_OUT), lambda i: (0, 0)),
            pl.BlockSpec((1, D_OUT), lambda i: (0, 0)),
        ],
        out_specs=pl.BlockSpec((BM, D_OUT), lambda i: (i, 0)),
        out_shape=jax.ShapeDtypeStruct((B, D_OUT), jnp.float32),
    )(self_feats, neigh_contrib, w1, b2d)


def kernel(nodes, neigh_index, features, w, b):
    B, DEG = neigh_index.shape
    N, D = features.shape
    CH = 128 // DEG
    neigh2d = neigh_index.reshape(B // CH, CH * DEG)
    w1 = w[:D]
    w2 = w[D:]
    D_OUT = w2.shape[1]
    w2g = w2.reshape(D, D_OUT // 32, 2, 16)
    w2lo_bf = w2g[:, :, 0, :].reshape(D, D_OUT // 2).astype(jnp.bfloat16)
    w2hi_bf = w2g[:, :, 1, :].reshape(D, D_OUT // 2).astype(jnp.bfloat16)
    p_packed = _tc_pack_table(features, w2lo_bf, w2hi_bf)
    self_feats, neigh_contrib = _sc_gather_mean(
        nodes, neigh2d, features, p_packed, B, DEG, D)
    return _tc_combine(self_feats, neigh_contrib, w1, b.reshape(1, -1))


# speedup vs baseline: 1.3812x; 1.0272x over previous
import functools

import jax
import jax.numpy as jnp
from jax import lax
from jax.experimental import pallas as pl
from jax.experimental.pallas import tpu as pltpu
from jax.experimental.pallas import tpu_sc as plsc

NC = 2
NS = 16
LANES = 16
NW = NC * NS


def _tc_pack_table(features, w2lo_bf, w2hi_bf):
    N, D = features.shape
    H = w2lo_bf.shape[1]
    BN = 1000

    def body(f_ref, wl_ref, wh_ref, o_ref):
        x = f_ref[...].astype(jnp.bfloat16)
        ylo = jnp.dot(x, wl_ref[...], preferred_element_type=jnp.float32)
        yhi = jnp.dot(x, wh_ref[...], preferred_element_type=jnp.float32)
        lo16 = jax.lax.bitcast_convert_type(
            ylo.astype(jnp.bfloat16), jnp.uint16).astype(jnp.uint32)
        hi16 = jax.lax.bitcast_convert_type(
            yhi.astype(jnp.bfloat16), jnp.uint16).astype(jnp.uint32)
        packed = lo16 | (hi16 << 16)
        o_ref[...] = jax.lax.bitcast_convert_type(packed, jnp.float32)

    return pl.pallas_call(
        body,
        grid=(N // BN,),
        in_specs=[
            pl.BlockSpec((BN, D), lambda i: (i, 0)),
            pl.BlockSpec((D, H), lambda i: (0, 0)),
            pl.BlockSpec((D, H), lambda i: (0, 0)),
        ],
        out_specs=pl.BlockSpec((BN, H), lambda i: (i, 0)),
        out_shape=jax.ShapeDtypeStruct((N, H), jnp.float32),
    )(features, w2lo_bf, w2hi_bf)


def _sc_gather_mean(nodes, neigh2d, features, p_packed, B, DEG, D):
    b_per_w = B // NW
    CH = 128 // DEG
    n_chunks = b_per_w // CH
    n_self = b_per_w // 128
    inv_deg = 1.0 / DEG
    H = D // 2
    nk = H // LANES

    mesh = plsc.VectorSubcoreMesh(
        core_axis_name="c", subcore_axis_name="s",
        num_cores=NC, num_subcores=NS)

    @functools.partial(
        pl.kernel,
        out_type=(
            jax.ShapeDtypeStruct((B, D), jnp.float32),
            jax.ShapeDtypeStruct((B, H), jnp.float32),
        ),
        mesh=mesh,
        scratch_types=[
            pltpu.VMEM((n_chunks, CH * DEG), jnp.int32),
            pltpu.VMEM((b_per_w,), jnp.int32),
            pltpu.VMEM((2, CH * DEG, H), jnp.float32),
            pltpu.VMEM((128, D), jnp.float32),
            pltpu.VMEM((CH, H), jnp.float32),
            pltpu.SemaphoreType.DMA,
            pltpu.SemaphoreType.DMA,
            pltpu.SemaphoreType.DMA,
        ],
    )
    def sc_kernel(nodes_hbm, nidx_hbm, feat_hbm, pk_hbm,
                  selfout_hbm, neighout_hbm,
                  idx_all, sidx, rows2, srows, acc_v, sem0, sem1, ssem):
        wid = lax.axis_index("s") * NC + lax.axis_index("c")
        base = wid * b_per_w
        cbase = wid * n_chunks

        pltpu.sync_copy(nidx_hbm.at[pl.ds(cbase, n_chunks)], idx_all)
        pltpu.sync_copy(nodes_hbm.at[pl.ds(base, b_per_w)], sidx)

        sems = (sem0, sem1)
        pltpu.async_copy(pk_hbm.at[idx_all.at[0]], rows2.at[0], sem0)
        pltpu.async_copy(pk_hbm.at[idx_all.at[1]], rows2.at[1], sem1)

        @pl.loop(0, n_self)
        def _self_loop(sc):
            pltpu.async_copy(
                feat_hbm.at[sidx.at[pl.ds(sc * 128, 128)]], srows, ssem
            ).wait()
            pltpu.sync_copy(
                srows, selfout_hbm.at[pl.ds(base + sc * 128, 128)])

        mask_hi = jnp.int32(-65536)

        @pl.loop(0, n_chunks, step=2)
        def _chunk_loop(c):
            for bsel in range(2):
                cc = c + bsel
                pltpu.make_async_copy(
                    pk_hbm.at[idx_all.at[0]], rows2.at[bsel], sems[bsel]
                ).wait()
                for r in range(CH):

                    def unpack_row(j):
                        out = []
                        for k in range(nk):
                            v = jax.lax.bitcast_convert_type(
                                rows2[bsel, j, pl.ds(k * LANES, LANES)],
                                jnp.int32)
                            e = jax.lax.bitcast_convert_type(
                                v << 16, jnp.float32)
                            o = jax.lax.bitcast_convert_type(
                                v & mask_hi, jnp.float32)
                            out.append((e, o))
                        return out

                    init = tuple(unpack_row(r * DEG))

                    @pl.loop(1, DEG, init_carry=init, unroll=4)
                    def _row_loop(j, carry):
                        row = unpack_row(r * DEG + j)
                        return tuple(
                            (ce + e, co + o)
                            for (ce, co), (e, o) in zip(carry, row))

                    blocks = []
                    for k in range(nk):
                        ce, co = _row_loop[k]
                        blocks.append(ce * inv_deg)
                        blocks.append(co * inv_deg)
                    half = jnp.int32(0x8000)
                    for m in range(nk):
                        lo_i = jax.lax.shift_right_logical(
                            jax.lax.bitcast_convert_type(
                                blocks[m], jnp.int32) + half, 16)
                        hi_i = (jax.lax.bitcast_convert_type(
                            blocks[m + nk], jnp.int32) + half) & mask_hi
                        acc_v[r, pl.ds(m * LANES, LANES)] = (
                            jax.lax.bitcast_convert_type(
                                lo_i | hi_i, jnp.float32))
                pltpu.sync_copy(
                    acc_v, neighout_hbm.at[pl.ds(base + cc * CH, CH)])

                @pl.when(cc + 2 < n_chunks)
                def _refill():
                    pltpu.async_copy(
                        pk_hbm.at[idx_all.at[cc + 2]], rows2.at[bsel],
                        sems[bsel])

    return sc_kernel(nodes, neigh2d, features, p_packed)


def _tc_combine(self_feats, neigh_packed, w1, b2d):
    B, D = self_feats.shape
    D_OUT = w1.shape[1]
    H = neigh_packed.shape[1]
    BM = 1024

    def body(s_ref, n_ref, w1_ref, b_ref, o_ref):
        x = jnp.dot(s_ref[...], w1_ref[...], preferred_element_type=jnp.float32)
        v = jax.lax.bitcast_convert_type(n_ref[...], jnp.int32)
        lo = jax.lax.bitcast_convert_type(v << 16, jnp.float32)
        hi = jax.lax.bitcast_convert_type(
            v & jnp.int32(-65536), jnp.float32)
        x = x + jnp.concatenate([lo, hi], axis=1)
        x = x + b_ref[...]
        x = jnp.maximum(x, 0.0)
        nrm = jnp.sqrt(jnp.sum(x * x, axis=1, keepdims=True))
        o_ref[...] = x / jnp.maximum(nrm, 1e-12)

    return pl.pallas_call(
        body,
        grid=(B // BM,),
        in_specs=[
            pl.BlockSpec((BM, D), lambda i: (i, 0)),
            pl.BlockSpec((BM, H), lambda i: (i, 0)),
            pl.BlockSpec((D, D_OUT), lambda i: (0, 0)),
            pl.BlockSpec((1, D_OUT), lambda i: (0, 0)),
        ],
        out_specs=pl.BlockSpec((BM, D_OUT), lambda i: (i, 0)),
        out_shape=jax.ShapeDtypeStruct((B, D_OUT), jnp.float32),
    )(self_feats, neigh_packed, w1, b2d)


def kernel(nodes, neigh_index, features, w, b):
    B, DEG = neigh_index.shape
    N, D = features.shape
    CH = 128 // DEG
    neigh2d = neigh_index.reshape(B // CH, CH * DEG)
    w1 = w[:D]
    w2 = w[D:]
    D_OUT = w2.shape[1]
    w2g = w2.reshape(D, D_OUT // 32, 2, 16)
    w2lo_bf = w2g[:, :, 0, :].reshape(D, D_OUT // 2).astype(jnp.bfloat16)
    w2hi_bf = w2g[:, :, 1, :].reshape(D, D_OUT // 2).astype(jnp.bfloat16)
    p_packed = _tc_pack_table(features, w2lo_bf, w2hi_bf)
    self_feats, neigh_contrib = _sc_gather_mean(
        nodes, neigh2d, features, p_packed, B, DEG, D)
    return _tc_combine(self_feats, neigh_contrib, w1, b.reshape(1, -1))
